# baseline (speedup 1.0000x reference)
import functools

import jax
import jax.numpy as jnp
from jax import lax
from jax.experimental import pallas as pl
from jax.experimental.pallas import tpu as pltpu
from jax.experimental.pallas import tpu_sc as plsc

B = 16384
K = 16
NC = 2
NS = 16
NW = NC * NS
BPW = B // NW
NCHUNK = 4
CHUNK = BPW // NCHUNK


def _sc_kernel(w_hbm, h_hbm, uidx_hbm, iidx_hbm, out_hbm,
               uidx_v, iidx_v, urows_v, vrows_v, out_v, sem):
    wid = lax.axis_index("c") * NS + lax.axis_index("s")

    pltpu.sync_copy(uidx_hbm.at[wid], uidx_v)
    pltpu.sync_copy(iidx_hbm.at[wid], iidx_v)

    copies = []
    for c in range(NCHUNK):
        copies.append(pltpu.async_copy(
            w_hbm.at[uidx_v.at[c]], urows_v.at[pl.ds(c * CHUNK, CHUNK)], sem))
        copies.append(pltpu.async_copy(
            h_hbm.at[iidx_v.at[c]], vrows_v.at[pl.ds(c * CHUNK, CHUNK)], sem))
    for cp in copies:
        cp.wait()

    lane = lax.iota(jnp.int32, 16)

    def group_body(g, _):
        r0 = g * 16
        acc = jnp.zeros((16,), jnp.float32)
        for j in range(16):
            u = urows_v[r0 + j, :]
            v = vrows_v[r0 + j, :]
            s = jnp.sum(u * v)
            acc = jnp.where(lane == j, s, acc)
        out_v[pl.ds(r0, 16)] = acc
        return _

    lax.fori_loop(0, BPW // 16, group_body, None)

    pltpu.sync_copy(out_v, out_hbm.at[pl.ds(wid * BPW, BPW)])


@jax.jit
def _run(w, h, uidx, iidx):
    mesh = plsc.VectorSubcoreMesh(core_axis_name="c", subcore_axis_name="s")
    fn = pl.kernel(
        _sc_kernel,
        mesh=mesh,
        compiler_params=pltpu.CompilerParams(
            needs_layout_passes=False, use_tc_tiling_on_sc=False),
        out_type=jax.ShapeDtypeStruct((B,), jnp.float32),
        scratch_types=[
            pltpu.VMEM((NCHUNK, CHUNK), jnp.int32),
            pltpu.VMEM((NCHUNK, CHUNK), jnp.int32),
            pltpu.VMEM((BPW, K), jnp.float32),
            pltpu.VMEM((BPW, K), jnp.float32),
            pltpu.VMEM((BPW,), jnp.float32),
            pltpu.SemaphoreType.DMA,
        ],
    )
    return fn(w, h, uidx, iidx)


def kernel(x, W, H):
    uidx = x[:, 0].reshape(NW, NCHUNK, CHUNK)
    iidx = x[:, 1].reshape(NW, NCHUNK, CHUNK)
    return _run(W, H, uidx, iidx)
